# speedup vs baseline: 5.8831x; 5.8831x over previous
import functools

import jax
import jax.numpy as jnp
from jax import lax
from jax.experimental import pallas as pl
from jax.experimental.pallas import tpu as pltpu
from jax.experimental.pallas import tpu_sc as plsc

N = 10000
E = 320000
D = 128
HD = 128
G = 64

NP = 10240
NW = 32
K = 128
C = 80
EP = NW * C * K
RPT = NP // 16
R = 1024


def _sc_mesh():
    return plsc.VectorSubcoreMesh(core_axis_name="c", subcore_axis_name="s")


def _sc_degree(dst_r):
    @functools.partial(
        pl.kernel,
        out_type=jax.ShapeDtypeStruct((2, NP, 16), jnp.float32),
        mesh=_sc_mesh(),
        scratch_types=[
            pltpu.VMEM((C, K), jnp.int32),
            pltpu.VMEM((K, 16), jnp.float32),
            pltpu.VMEM((16, 16), jnp.float32),
            pltpu.VMEM_SHARED((NP, 16), jnp.float32),
            pltpu.SemaphoreType.DMA,
        ],
    )
    def k(dst_hbm, out_hbm, dst_v, ones_v, zrow_v, acc, sem):
        c = lax.axis_index("c")
        s = lax.axis_index("s")
        wid = c * 16 + s
        z16 = jnp.zeros((16,), jnp.float32)
        o16 = jnp.ones((16,), jnp.float32)
        for i in range(16):
            zrow_v[i, :] = z16
        for i in range(K):
            ones_v[i, :] = o16
        base = s * RPT

        def zero_slice(r, carry):
            pltpu.sync_copy(zrow_v, acc.at[pl.ds(base + r * 16, 16)])
            return carry

        lax.fori_loop(0, RPT // 16, zero_slice, 0)
        plsc.subcore_barrier()

        pltpu.sync_copy(dst_hbm.at[wid], dst_v)

        def hist(j, carry):
            pltpu.sync_copy(ones_v, acc.at[dst_v.at[j]], add=True)
            return carry

        lax.fori_loop(0, C, hist, 0)
        plsc.subcore_barrier()
        pltpu.sync_copy(acc.at[pl.ds(base, RPT)],
                        out_hbm.at[c].at[pl.ds(base, RPT)])

    return k(dst_r)


def _sc_spmm(src_r, dst_r, hw2):
    @functools.partial(
        pl.kernel,
        out_type=jax.ShapeDtypeStruct((2, NP, 128), jnp.float32),
        mesh=_sc_mesh(),
        scratch_types=[
            pltpu.VMEM((C, K), jnp.int32),
            pltpu.VMEM((C, K), jnp.int32),
            pltpu.VMEM((K, 128), jnp.float32),
            pltpu.VMEM((16, 128), jnp.float32),
            pltpu.VMEM_SHARED((NP, 128), jnp.float32),
            pltpu.SemaphoreType.DMA,
        ],
    )
    def k(src_hbm, dst_hbm, hw2_hbm, out_hbm, src_v, dst_v, rows_v, zrow_v,
          acc, sem):
        c = lax.axis_index("c")
        s = lax.axis_index("s")
        wid = c * 16 + s
        z16 = jnp.zeros((16,), jnp.float32)
        for i in range(16):
            for j in range(8):
                zrow_v[i, pl.ds(j * 16, 16)] = z16
        base = s * RPT

        def zero_slice(r, carry):
            pltpu.sync_copy(zrow_v, acc.at[pl.ds(base + r * 16, 16)])
            return carry

        lax.fori_loop(0, RPT // 16, zero_slice, 0)
        plsc.subcore_barrier()

        pltpu.sync_copy(src_hbm.at[wid], src_v)
        pltpu.sync_copy(dst_hbm.at[wid], dst_v)

        def edge_chunk(j, carry):
            pltpu.async_copy(hw2_hbm.at[src_v.at[j]], rows_v, sem).wait()
            pltpu.sync_copy(rows_v, acc.at[dst_v.at[j]], add=True)
            return carry

        lax.fori_loop(0, C, edge_chunk, 0)
        plsc.subcore_barrier()
        pltpu.sync_copy(acc.at[pl.ds(base, RPT)],
                        out_hbm.at[c].at[pl.ds(base, RPT)])

    return k(src_r, dst_r, hw2)


def _dinv_block(deg_ref):
    return lax.rsqrt(deg_ref[0, :, 0:1] + deg_ref[1, :, 0:1] + 1.0)


def _tc_first(degp, x_pad, W1):
    def body(d_ref, x_ref, w_ref, o_ref):
        dinv = _dinv_block(d_ref)
        o_ref[...] = jnp.dot(x_ref[...], w_ref[...],
                             preferred_element_type=jnp.float32) * dinv

    return pl.pallas_call(
        body,
        grid=(NP // R,),
        in_specs=[
            pl.BlockSpec((2, R, 16), lambda i: (0, i, 0)),
            pl.BlockSpec((R, D), lambda i: (i, 0)),
            pl.BlockSpec((D, HD), lambda i: (0, 0)),
        ],
        out_specs=pl.BlockSpec((R, HD), lambda i: (i, 0)),
        out_shape=jax.ShapeDtypeStruct((NP, HD), jnp.float32),
    )(degp, x_pad, W1)


def _tc_mid(S, hw2p, degp, b, Wn):
    def body(s_ref, hw_ref, d_ref, b_ref, w_ref, o_ref):
        dinv = _dinv_block(d_ref)
        h = jnp.maximum(
            dinv * (s_ref[0] + s_ref[1] + hw_ref[...]) + b_ref[...], 0.0)
        o_ref[...] = jnp.dot(h, w_ref[...],
                             preferred_element_type=jnp.float32) * dinv

    return pl.pallas_call(
        body,
        grid=(NP // R,),
        in_specs=[
            pl.BlockSpec((2, R, HD), lambda i: (0, i, 0)),
            pl.BlockSpec((R, HD), lambda i: (i, 0)),
            pl.BlockSpec((2, R, 16), lambda i: (0, i, 0)),
            pl.BlockSpec((1, HD), lambda i: (0, 0)),
            pl.BlockSpec((HD, HD), lambda i: (0, 0)),
        ],
        out_specs=pl.BlockSpec((R, HD), lambda i: (i, 0)),
        out_shape=jax.ShapeDtypeStruct((NP, HD), jnp.float32),
    )(S, hw2p, degp, b, Wn)


def _tc_last_pool(S, hw2p, degp, b, batch_r):
    nblk = NP // R

    def body(s_ref, hw_ref, d_ref, b_ref, bt_ref, o_ref, sums, counts):
        i = pl.program_id(0)
        dinv = _dinv_block(d_ref)
        h = jnp.maximum(
            dinv * (s_ref[0] + s_ref[1] + hw_ref[...]) + b_ref[...], 0.0)
        bb = bt_ref[0, 0, :]
        gids = lax.broadcasted_iota(jnp.int32, (G, R), 0)
        oh = (gids == bb[None, :]).astype(jnp.float32)

        @pl.when(i == 0)
        def _():
            sums[...] = jnp.zeros((G, HD), jnp.float32)
            counts[...] = jnp.zeros((G, HD), jnp.float32)

        sums[...] += jnp.dot(oh, h, preferred_element_type=jnp.float32)
        counts[...] += jnp.broadcast_to(
            jnp.sum(oh, axis=1, keepdims=True), (G, HD))

        @pl.when(i == nblk - 1)
        def _():
            o_ref[...] = sums[...] / jnp.maximum(counts[...], 1.0)

    return pl.pallas_call(
        body,
        grid=(nblk,),
        in_specs=[
            pl.BlockSpec((2, R, HD), lambda i: (0, i, 0)),
            pl.BlockSpec((R, HD), lambda i: (i, 0)),
            pl.BlockSpec((2, R, 16), lambda i: (0, i, 0)),
            pl.BlockSpec((1, HD), lambda i: (0, 0)),
            pl.BlockSpec((1, 1, R), lambda i: (i, 0, 0)),
        ],
        out_specs=pl.BlockSpec((G, HD), lambda i: (0, 0)),
        out_shape=jax.ShapeDtypeStruct((G, HD), jnp.float32),
        scratch_shapes=[
            pltpu.VMEM((G, HD), jnp.float32),
            pltpu.VMEM((G, HD), jnp.float32),
        ],
    )(S, hw2p, degp, b, batch_r)


def kernel(x, edge_index, edge_attr, batch, W1, b1, W2, b2, W3, b3):
    src = edge_index[0]
    dst = edge_index[1]
    pad_e = EP - E
    src_r = jnp.concatenate(
        [src, jnp.zeros((pad_e,), jnp.int32)]).reshape(NW, C, K)
    dst_r = jnp.concatenate(
        [dst, jnp.full((pad_e,), NP - 1, jnp.int32)]).reshape(NW, C, K)
    x_pad = jnp.concatenate([x, jnp.zeros((NP - N, D), jnp.float32)], axis=0)
    batch_r = jnp.concatenate(
        [batch, jnp.full((NP - N,), G, jnp.int32)]).reshape(NP // R, 1, R)

    degp = _sc_degree(dst_r)
    hw2 = _tc_first(degp, x_pad, W1)
    S = _sc_spmm(src_r, dst_r, hw2)
    hw2 = _tc_mid(S, hw2, degp, b1.reshape(1, HD), W2)
    S = _sc_spmm(src_r, dst_r, hw2)
    hw2 = _tc_mid(S, hw2, degp, b2.reshape(1, HD), W3)
    S = _sc_spmm(src_r, dst_r, hw2)
    return _tc_last_pool(S, hw2, degp, b3.reshape(1, HD), batch_r)


# speedup vs baseline: 23.2697x; 3.9553x over previous
import functools

import jax
import jax.numpy as jnp
from jax import lax
from jax.experimental import pallas as pl
from jax.experimental.pallas import tpu as pltpu
from jax.experimental.pallas import tpu_sc as plsc

N = 10000
E = 320000
D = 128
HD = 128
G = 64

NP = 10240
NW = 32
K = 128
C = 80
EP = NW * C * K
RPT = NP // 16
R = 1024


def _sc_mesh():
    return plsc.VectorSubcoreMesh(core_axis_name="c", subcore_axis_name="s")


def _sc_degree(dst_r):
    @functools.partial(
        pl.kernel,
        out_type=jax.ShapeDtypeStruct((2, NP, 16), jnp.float32),
        mesh=_sc_mesh(),
        scratch_types=[
            pltpu.VMEM((C, K), jnp.int32),
            pltpu.VMEM((K, 16), jnp.float32),
            pltpu.VMEM((16, 16), jnp.float32),
            pltpu.VMEM_SHARED((NP, 16), jnp.float32),
            pltpu.SemaphoreType.DMA,
        ],
    )
    def k(dst_hbm, out_hbm, dst_v, ones_v, zrow_v, acc, sem):
        c = lax.axis_index("c")
        s = lax.axis_index("s")
        wid = c * 16 + s
        z16 = jnp.zeros((16,), jnp.float32)
        o16 = jnp.ones((16,), jnp.float32)
        for i in range(16):
            zrow_v[i, :] = z16
        for i in range(K):
            ones_v[i, :] = o16
        base = s * RPT

        def zero_slice(r, carry):
            pltpu.sync_copy(zrow_v, acc.at[pl.ds(base + r * 16, 16)])
            return carry

        lax.fori_loop(0, RPT // 16, zero_slice, 0)
        plsc.subcore_barrier()

        pltpu.sync_copy(dst_hbm.at[wid], dst_v)

        def hist(j, carry):
            pltpu.sync_copy(ones_v, acc.at[dst_v.at[j]], add=True)
            return carry

        lax.fori_loop(0, C, hist, 0)
        plsc.subcore_barrier()
        pltpu.sync_copy(acc.at[pl.ds(base, RPT)],
                        out_hbm.at[c].at[pl.ds(base, RPT)])

    return k(dst_r)


def _sc_spmm(idx_r, hw2):
    @functools.partial(
        pl.kernel,
        out_type=jax.ShapeDtypeStruct((2, NP, 128), jnp.float32),
        mesh=_sc_mesh(),
        scratch_types=[
            pltpu.VMEM((4, 2, K), jnp.int32),
            pltpu.VMEM((K, 128), jnp.float32),
            pltpu.VMEM((K, 128), jnp.float32),
            pltpu.VMEM((64, 128), jnp.float32),
            pltpu.VMEM_SHARED((NP, 128), jnp.float32),
            pltpu.SemaphoreType.DMA,
            pltpu.SemaphoreType.DMA,
            pltpu.SemaphoreType.DMA,
            pltpu.SemaphoreType.DMA,
            pltpu.SemaphoreType.DMA,
            pltpu.SemaphoreType.DMA,
        ],
    )
    def k(idx_hbm, hw2_hbm, out_hbm, ibuf, rows0_v, rows1_v, zbuf_v, acc,
          gsem0, gsem1, isem0, isem1, isem2, isem3):
        c = lax.axis_index("c")
        s = lax.axis_index("s")
        wid = c * 16 + s
        z16 = jnp.zeros((16,), jnp.float32)

        def zero_row(i, carry):
            for j in range(8):
                zbuf_v[i, pl.ds(j * 16, 16)] = z16
            return carry

        lax.fori_loop(0, 64, zero_row, 0)
        base = s * RPT

        def zero_slice(r, carry):
            pltpu.sync_copy(zbuf_v, acc.at[pl.ds(base + r * 64, 64)])
            return carry

        lax.fori_loop(0, RPT // 64, zero_slice, 0)
        plsc.subcore_barrier()

        rows = (rows0_v, rows1_v)
        gsems = (gsem0, gsem1)
        isems = (isem0, isem1, isem2, isem3)
        me = idx_hbm.at[wid]

        for j in range(4):
            pltpu.async_copy(me.at[j], ibuf.at[j], isems[j])
        for j in range(2):
            pltpu.make_async_copy(me.at[j], ibuf.at[j], isems[j]).wait()
            pltpu.async_copy(hw2_hbm.at[ibuf.at[j, 0]], rows[j], gsems[j])

        def step(i, carry):
            for b in range(4):
                j = 4 * i + b
                bb = b % 2
                s2 = (b + 2) % 4
                pltpu.make_async_copy(
                    hw2_hbm.at[ibuf.at[b, 0]], rows[bb], gsems[bb]).wait()
                pltpu.sync_copy(rows[bb], acc.at[ibuf.at[b, 1]], add=True)
                pltpu.make_async_copy(
                    me.at[j + 2], ibuf.at[s2], isems[s2]).wait()
                pltpu.async_copy(
                    hw2_hbm.at[ibuf.at[s2, 0]], rows[bb], gsems[bb])
                pltpu.async_copy(me.at[j + 4], ibuf.at[b], isems[b])
            return carry

        lax.fori_loop(0, (C - 8) // 4, step, 0)

        for j in range(C - 8, C):
            b = j % 4
            bb = j % 2
            s2 = (b + 2) % 4
            pltpu.make_async_copy(
                hw2_hbm.at[ibuf.at[b, 0]], rows[bb], gsems[bb]).wait()
            pltpu.sync_copy(rows[bb], acc.at[ibuf.at[b, 1]], add=True)
            if j + 2 < C:
                pltpu.make_async_copy(
                    me.at[j + 2], ibuf.at[s2], isems[s2]).wait()
                pltpu.async_copy(
                    hw2_hbm.at[ibuf.at[s2, 0]], rows[bb], gsems[bb])
            if j + 4 < C:
                pltpu.async_copy(me.at[j + 4], ibuf.at[b], isems[b])
        plsc.subcore_barrier()
        pltpu.sync_copy(acc.at[pl.ds(base, RPT)],
                        out_hbm.at[c].at[pl.ds(base, RPT)])

    return k(idx_r, hw2)


def _dinv_block(deg_ref):
    return lax.rsqrt(deg_ref[0, :, 0:1] + deg_ref[1, :, 0:1] + 1.0)


def _tc_first(degp, x_pad, W1):
    def body(d_ref, x_ref, w_ref, o_ref):
        dinv = _dinv_block(d_ref)
        o_ref[...] = jnp.dot(x_ref[...], w_ref[...],
                             preferred_element_type=jnp.float32) * dinv

    return pl.pallas_call(
        body,
        grid=(NP // R,),
        in_specs=[
            pl.BlockSpec((2, R, 16), lambda i: (0, i, 0)),
            pl.BlockSpec((R, D), lambda i: (i, 0)),
            pl.BlockSpec((D, HD), lambda i: (0, 0)),
        ],
        out_specs=pl.BlockSpec((R, HD), lambda i: (i, 0)),
        out_shape=jax.ShapeDtypeStruct((NP, HD), jnp.float32),
    )(degp, x_pad, W1)


def _tc_mid(S, hw2p, degp, b, Wn):
    def body(s_ref, hw_ref, d_ref, b_ref, w_ref, o_ref):
        dinv = _dinv_block(d_ref)
        h = jnp.maximum(
            dinv * (s_ref[0] + s_ref[1] + hw_ref[...]) + b_ref[...], 0.0)
        o_ref[...] = jnp.dot(h, w_ref[...],
                             preferred_element_type=jnp.float32) * dinv

    return pl.pallas_call(
        body,
        grid=(NP // R,),
        in_specs=[
            pl.BlockSpec((2, R, HD), lambda i: (0, i, 0)),
            pl.BlockSpec((R, HD), lambda i: (i, 0)),
            pl.BlockSpec((2, R, 16), lambda i: (0, i, 0)),
            pl.BlockSpec((1, HD), lambda i: (0, 0)),
            pl.BlockSpec((HD, HD), lambda i: (0, 0)),
        ],
        out_specs=pl.BlockSpec((R, HD), lambda i: (i, 0)),
        out_shape=jax.ShapeDtypeStruct((NP, HD), jnp.float32),
    )(S, hw2p, degp, b, Wn)


def _tc_last_pool(S, hw2p, degp, b, batch_r):
    nblk = NP // R

    def body(s_ref, hw_ref, d_ref, b_ref, bt_ref, o_ref, sums, counts):
        i = pl.program_id(0)
        dinv = _dinv_block(d_ref)
        h = jnp.maximum(
            dinv * (s_ref[0] + s_ref[1] + hw_ref[...]) + b_ref[...], 0.0)
        bb = bt_ref[0, 0, :]
        gids = lax.broadcasted_iota(jnp.int32, (G, R), 0)
        oh = (gids == bb[None, :]).astype(jnp.float32)

        @pl.when(i == 0)
        def _():
            sums[...] = jnp.zeros((G, HD), jnp.float32)
            counts[...] = jnp.zeros((G, HD), jnp.float32)

        sums[...] += jnp.dot(oh, h, preferred_element_type=jnp.float32)
        counts[...] += jnp.broadcast_to(
            jnp.sum(oh, axis=1, keepdims=True), (G, HD))

        @pl.when(i == nblk - 1)
        def _():
            o_ref[...] = sums[...] / jnp.maximum(counts[...], 1.0)

    return pl.pallas_call(
        body,
        grid=(nblk,),
        in_specs=[
            pl.BlockSpec((2, R, HD), lambda i: (0, i, 0)),
            pl.BlockSpec((R, HD), lambda i: (i, 0)),
            pl.BlockSpec((2, R, 16), lambda i: (0, i, 0)),
            pl.BlockSpec((1, HD), lambda i: (0, 0)),
            pl.BlockSpec((1, 1, R), lambda i: (i, 0, 0)),
        ],
        out_specs=pl.BlockSpec((G, HD), lambda i: (0, 0)),
        out_shape=jax.ShapeDtypeStruct((G, HD), jnp.float32),
        scratch_shapes=[
            pltpu.VMEM((G, HD), jnp.float32),
            pltpu.VMEM((G, HD), jnp.float32),
        ],
    )(S, hw2p, degp, b, batch_r)


def kernel(x, edge_index, edge_attr, batch, W1, b1, W2, b2, W3, b3):
    src = edge_index[0]
    dst = edge_index[1]
    pad_e = EP - E
    cyc = jax.lax.iota(jnp.int32, pad_e)
    src_r = jnp.concatenate([src, cyc % N]).reshape(NW, C, K)
    dst_r = jnp.concatenate([dst, N + cyc % (NP - N)]).reshape(NW, C, K)
    idx_r = jnp.stack([src_r, dst_r], axis=2)
    x_pad = jnp.concatenate([x, jnp.zeros((NP - N, D), jnp.float32)], axis=0)
    batch_r = jnp.concatenate(
        [batch, jnp.full((NP - N,), G, jnp.int32)]).reshape(NP // R, 1, R)

    degp = _sc_degree(dst_r)
    hw2 = _tc_first(degp, x_pad, W1)
    S = _sc_spmm(idx_r, hw2)
    hw2 = _tc_mid(S, hw2, degp, b1.reshape(1, HD), W2)
    S = _sc_spmm(idx_r, hw2)
    hw2 = _tc_mid(S, hw2, degp, b2.reshape(1, HD), W3)
    S = _sc_spmm(idx_r, hw2)
    return _tc_last_pool(S, hw2, degp, b3.reshape(1, HD), batch_r)
